# baseline (speedup 1.0000x reference)
import functools

import jax
import jax.numpy as jnp
from jax.experimental import pallas as pl
from jax.experimental.pallas import tpu as pltpu

B, L, C, P = 64, 512, 768, 196
PP = 200
S = 8
G = B // S


def _aet_body(bbox_ref, mask_ref, text_ref, image_ref, out_ref):
    @pl.when(pl.program_id(0) == 0)
    def _():
        out_ref[...] = jnp.zeros_like(out_ref)

    losses = []
    for i in range(S):
        dt = jnp.transpose(bbox_ref[i] // 72)
        x0 = dt[:, 0:1]
        y0 = dt[:, 1:2]
        x1 = dt[:, 2:3]
        y1 = dt[:, 3:4]
        mask_t = jnp.transpose(mask_ref[i])
        valid = (x0 == x1) & (y0 == y1) & (mask_t != 0)
        lab = jnp.where(valid, y0 * 14 + x0, -100)

        cols = jax.lax.broadcasted_iota(jnp.int32, (L, P), 1)
        eq = cols == lab
        oh_t = eq.astype(jnp.bfloat16)
        text_i = text_ref[i * L:(i + 1) * L, :].astype(jnp.bfloat16)
        sums_t = jax.lax.dot_general(
            text_i, oh_t, (((0,), (0,)), ((), ())),
            preferred_element_type=jnp.float32)
        cnts = jnp.sum(eq.astype(jnp.float32), axis=0, keepdims=True)
        tpe_t = (sums_t * (1.0 / jnp.maximum(cnts, 1.0))).astype(jnp.bfloat16)

        img_i = image_ref[i]
        m = jax.lax.dot_general(
            img_i, tpe_t, (((1,), (0,)), ((), ())),
            preferred_element_type=jnp.float32)

        mr = jnp.max(m, axis=1, keepdims=True)
        lse_r = jnp.log(jnp.sum(jnp.exp(m - mr), axis=1, keepdims=True)) + mr
        mc = jnp.max(m, axis=0, keepdims=True)
        lse_c = jnp.log(jnp.sum(jnp.exp(m - mc), axis=0, keepdims=True)) + mc

        ii = jax.lax.broadcasted_iota(jnp.int32, (P, P), 0)
        jj = jax.lax.broadcasted_iota(jnp.int32, (P, P), 1)
        diag_sum = jnp.sum(jnp.where(ii == jj, m, 0.0))

        loss = ((jnp.sum(lse_r) + jnp.sum(lse_c)) * 0.5 - diag_sum) * (1.0 / P)
        losses.append(loss)
    total = sum(losses[1:], losses[0])
    out_ref[...] = out_ref[...] + total.reshape(1, 1)


@functools.partial(jax.jit, static_argnames=())
def kernel(text_embeds, image_patch_embedding, bbox, attention_mask):
    bbox_t = jnp.transpose(bbox.astype(jnp.int32), (0, 2, 1))
    mask3 = attention_mask.astype(jnp.int32).reshape(B, 1, L)
    text2 = text_embeds.reshape(B * L, C)
    img_bf = image_patch_embedding.astype(jnp.bfloat16)

    total = pl.pallas_call(
        _aet_body,
        out_shape=jax.ShapeDtypeStruct((1, 1), jnp.float32),
        grid=(G,),
        in_specs=[
            pl.BlockSpec((S, 4, L), lambda b: (b, 0, 0)),
            pl.BlockSpec((S, 1, L), lambda b: (b, 0, 0)),
            pl.BlockSpec((S * L, C), lambda b: (b, 0)),
            pl.BlockSpec((S, P, C), lambda b: (b, 0, 0)),
        ],
        out_specs=pl.BlockSpec((1, 1), lambda b: (0, 0)),
        compiler_params=pltpu.CompilerParams(
            dimension_semantics=("arbitrary",),
        ),
        name="aet_loss",
    )(bbox_t, mask3, text2, img_bf)

    return total[0, 0] * (1.0 / B)


# speedup vs baseline: 1.7580x; 1.7580x over previous
import functools

import jax
import jax.numpy as jnp
from jax.experimental import pallas as pl
from jax.experimental.pallas import tpu as pltpu

B, L, C, P = 64, 512, 768, 196
PP = 200
S = 8
G = B // S


def _aet_body(bbox_ref, mask_ref, text_ref, image_ref, out_ref):
    @pl.when(pl.program_id(0) == 0)
    def _():
        out_ref[...] = jnp.zeros_like(out_ref)

    losses = []
    for i in range(S):
        d = bbox_ref[i] // 72
        x0 = d[0:1, :]
        y0 = d[1:2, :]
        x1 = d[2:3, :]
        y1 = d[3:4, :]
        valid = (x0 == x1) & (y0 == y1) & (mask_ref[i] != 0)
        lab = jnp.where(valid, y0 * 14 + x0, -100)

        rows = jax.lax.broadcasted_iota(jnp.int32, (P, L), 0)
        eq = rows == lab
        oh = eq.astype(jnp.bfloat16)
        text_i = text_ref[i * L:(i + 1) * L, :].astype(jnp.bfloat16)
        sums = jnp.dot(oh, text_i, preferred_element_type=jnp.float32)
        cnts = jnp.sum(eq.astype(jnp.float32), axis=1, keepdims=True)
        tpe = (sums * (1.0 / jnp.maximum(cnts, 1.0))).astype(jnp.bfloat16)

        img_i = image_ref[i]
        m = jax.lax.dot_general(
            img_i, tpe, (((1,), (1,)), ((), ())),
            preferred_element_type=jnp.float32)

        mr = jnp.max(m, axis=1, keepdims=True)
        lse_r = jnp.log(jnp.sum(jnp.exp(m - mr), axis=1, keepdims=True)) + mr
        mc = jnp.max(m, axis=0, keepdims=True)
        lse_c = jnp.log(jnp.sum(jnp.exp(m - mc), axis=0, keepdims=True)) + mc

        ii = jax.lax.broadcasted_iota(jnp.int32, (P, P), 0)
        jj = jax.lax.broadcasted_iota(jnp.int32, (P, P), 1)
        diag_sum = jnp.sum(jnp.where(ii == jj, m, 0.0))

        loss = ((jnp.sum(lse_r) + jnp.sum(lse_c)) * 0.5 - diag_sum) * (1.0 / P)
        losses.append(loss)
    total = sum(losses[1:], losses[0])
    out_ref[...] = out_ref[...] + total.reshape(1, 1)


@functools.partial(jax.jit, static_argnames=())
def kernel(text_embeds, image_patch_embedding, bbox, attention_mask):
    bbox_t = jnp.transpose(bbox.astype(jnp.int32), (0, 2, 1))
    mask3 = attention_mask.astype(jnp.int32).reshape(B, 1, L)
    text2 = text_embeds.reshape(B * L, C)
    img_bf = image_patch_embedding.astype(jnp.bfloat16)

    total = pl.pallas_call(
        _aet_body,
        out_shape=jax.ShapeDtypeStruct((1, 1), jnp.float32),
        grid=(G,),
        in_specs=[
            pl.BlockSpec((S, 4, L), lambda b: (b, 0, 0)),
            pl.BlockSpec((S, 1, L), lambda b: (b, 0, 0)),
            pl.BlockSpec((S * L, C), lambda b: (b, 0)),
            pl.BlockSpec((S, P, C), lambda b: (b, 0, 0)),
        ],
        out_specs=pl.BlockSpec((1, 1), lambda b: (0, 0)),
        compiler_params=pltpu.CompilerParams(
            dimension_semantics=("arbitrary",),
        ),
        name="aet_loss",
    )(bbox_t, mask3, text2, img_bf)

    return total[0, 0] * (1.0 / B)
